# baseline (speedup 1.0000x reference)
import numpy as np
import jax
import jax.numpy as jnp
from jax.experimental import pallas as pl

_MAX_WAVELENGTH = 10000.0


def _sine_pe_np(seq_len: int, dim: int) -> np.ndarray:
    position = np.arange(seq_len, dtype=np.float64)
    min_freq = 1.0 / _MAX_WAVELENGTH
    timescales = np.power(
        min_freq,
        (2 * (np.arange(dim) // 2)).astype(np.float64) / float(dim),
    )
    angles = position[:, None] * timescales[None, :]
    cos_mask = (np.arange(dim) % 2).astype(np.float64)
    pe = np.sin(angles) * (1.0 - cos_mask) + np.cos(angles) * cos_mask
    return pe.astype(np.float32)


def _add_body(x_ref, pe_ref, o_ref):
    o_ref[...] = x_ref[...] + pe_ref[...]


def kernel(inputs, table):
    batch, seq_len, dim = inputs.shape
    row = seq_len * dim
    pe = jnp.asarray(_sine_pe_np(seq_len, dim).reshape(1, row))

    x = inputs.reshape(batch, row)
    bb = 256
    grid = (batch // bb,)
    out = pl.pallas_call(
        _add_body,
        grid=grid,
        in_specs=[
            pl.BlockSpec((bb, row), lambda i: (i, 0)),
            pl.BlockSpec((1, row), lambda i: (0, 0)),
        ],
        out_specs=pl.BlockSpec((bb, row), lambda i: (i, 0)),
        out_shape=jax.ShapeDtypeStruct((batch, row), jnp.float32),
    )(x, pe)
    return out.reshape(batch, seq_len, dim)


# speedup vs baseline: 1.4430x; 1.4430x over previous
import numpy as np
import jax
import jax.numpy as jnp
from jax.experimental import pallas as pl

_MAX_WAVELENGTH = 10000.0


def _sine_pe_np(seq_len: int, dim: int) -> np.ndarray:
    position = np.arange(seq_len, dtype=np.float64)
    min_freq = 1.0 / _MAX_WAVELENGTH
    timescales = np.power(
        min_freq,
        (2 * (np.arange(dim) // 2)).astype(np.float64) / float(dim),
    )
    angles = position[:, None] * timescales[None, :]
    cos_mask = (np.arange(dim) % 2).astype(np.float64)
    pe = np.sin(angles) * (1.0 - cos_mask) + np.cos(angles) * cos_mask
    return pe.astype(np.float32)


def _add_body(x_ref, pe_ref, o_ref):
    o_ref[...] = x_ref[...] + pe_ref[...]


def kernel(inputs, table):
    batch, seq_len, dim = inputs.shape
    pe = jnp.asarray(_sine_pe_np(seq_len, dim)[None])

    bb = 256
    grid = (batch // bb,)
    out = pl.pallas_call(
        _add_body,
        grid=grid,
        in_specs=[
            pl.BlockSpec((bb, seq_len, dim), lambda i: (i, 0, 0)),
            pl.BlockSpec((1, seq_len, dim), lambda i: (0, 0, 0)),
        ],
        out_specs=pl.BlockSpec((bb, seq_len, dim), lambda i: (i, 0, 0)),
        out_shape=jax.ShapeDtypeStruct((batch, seq_len, dim), jnp.float32),
    )(inputs, pe)
    return out
